# baseline (speedup 1.0000x reference)
import jax
import jax.numpy as jnp
from jax.experimental import pallas as pl
from jax.experimental.pallas import tpu as pltpu

N_TOKENS = 16384
D_IN = 4096
HIDDEN = 64
N_EXPERTS = 64
BT = 1024
KSPLIT = 4
KC = D_IN // KSPLIT


def _router_block(*refs):
    feat_refs = refs[:KSPLIT]
    w1_refs = refs[KSPLIT:2 * KSPLIT]
    b1_ref, w2_ref, b2_ref, w3_ref, b3_ref, hard_ref, probs_ref = refs[2 * KSPLIT:]
    h = jnp.dot(feat_refs[0][...], w1_refs[0][...],
                preferred_element_type=jnp.float32)
    for k in range(1, KSPLIT):
        h = h + jnp.dot(feat_refs[k][...], w1_refs[k][...],
                        preferred_element_type=jnp.float32)
    h = jnp.maximum(h + b1_ref[...], 0.0)
    h = jnp.dot(h, w2_ref[...], preferred_element_type=jnp.float32)
    h = jnp.maximum(h + b2_ref[...], 0.0)
    logits = jnp.dot(h, w3_ref[...], preferred_element_type=jnp.float32)
    logits = logits + b3_ref[...]
    m = jnp.max(logits, axis=-1, keepdims=True)
    e = jnp.exp(logits - m)
    probs = e / jnp.sum(e, axis=-1, keepdims=True)
    probs_ref[...] = probs
    idx = jnp.argmax(probs, axis=-1)
    lane = jax.lax.broadcasted_iota(jnp.int32, probs.shape, 1)
    hard_ref[...] = jnp.where(lane == idx[:, None], 1.0, 0.0).astype(jnp.float32)


@jax.jit
def kernel(feat, W1, b1, W2, b2, W3, b3):
    b1r = b1.reshape(1, HIDDEN)
    b2r = b2.reshape(1, HIDDEN)
    b3r = b3.reshape(1, N_EXPERTS)
    grid = (N_TOKENS // BT,)

    def feat_spec(k):
        return pl.BlockSpec((BT, KC), lambda i, _k=k: (i, _k))

    in_specs = (
        [feat_spec(k) for k in range(KSPLIT)]
        + [pl.BlockSpec((KC, HIDDEN), lambda i: (0, 0)) for _ in range(KSPLIT)]
        + [
            pl.BlockSpec((1, HIDDEN), lambda i: (0, 0)),
            pl.BlockSpec((HIDDEN, HIDDEN), lambda i: (0, 0)),
            pl.BlockSpec((1, HIDDEN), lambda i: (0, 0)),
            pl.BlockSpec((HIDDEN, N_EXPERTS), lambda i: (0, 0)),
            pl.BlockSpec((1, N_EXPERTS), lambda i: (0, 0)),
        ]
    )
    w1_chunks = [W1[k * KC:(k + 1) * KC] for k in range(KSPLIT)]
    out = pl.pallas_call(
        _router_block,
        grid=grid,
        in_specs=in_specs,
        out_specs=[
            pl.BlockSpec((BT, N_EXPERTS), lambda i: (i, 0)),
            pl.BlockSpec((BT, N_EXPERTS), lambda i: (i, 0)),
        ],
        out_shape=[
            jax.ShapeDtypeStruct((N_TOKENS, N_EXPERTS), jnp.float32),
            jax.ShapeDtypeStruct((N_TOKENS, N_EXPERTS), jnp.float32),
        ],
        compiler_params=pltpu.CompilerParams(
            dimension_semantics=("arbitrary",),
        ),
    )(*([feat] * KSPLIT), *w1_chunks, b1r, W2, b2r, W3, b3r)
    return out[0], out[1]
